# baseline (speedup 1.0000x reference)
import functools

import jax
import jax.numpy as jnp
from jax import lax
from jax.experimental import pallas as pl
from jax.experimental.pallas import tpu as pltpu
from jax.experimental.pallas import tpu_sc as plsc

_B, _P, _C = 1024, 200, 128
_NC, _NS = 2, 16
_NW = _NC * _NS
_BPW = _B // _NW
_LANES = 16
_PCHUNKS = (_P + _LANES - 1) // _LANES
_HALF = 104
_PT = 2
_ROWS_PER_B = (_C // 8) * _PT * 8
_CB = _C // _LANES
_TROWS = 128


def _body(xp_hbm, mp_hbm, table_hbm, out_hbm,
          xf_all, m_all, f_v, rows_v0, rows_v1, t_v0, t_v1, sem_g, sem_f):
  rows_v = (rows_v0, rows_v1)
  t_v = (t_v0, t_v1)
  cid = lax.axis_index("c")
  sid = lax.axis_index("s")
  wid = sid * _NC + cid
  wbase = wid * _BPW

  lanes = lax.iota(jnp.int32, _LANES)
  zeros = jnp.zeros((_LANES,), jnp.float32)

  pltpu.sync_copy(xp_hbm.at[pl.ds(wbase * _P, _BPW * _P)],
                  xf_all.at[pl.ds(0, _BPW * _P)])
  pltpu.sync_copy(mp_hbm.at[pl.ds(wbase * _P, _BPW * _P)],
                  m_all.at[pl.ds(0, _BPW * _P)])

  def issue_gather(i, k):
    off = i * _P
    pltpu.async_copy(table_hbm.at[xf_all.at[pl.ds(off, _HALF)]],
                     rows_v[k].at[pl.ds(0, _HALF)], sem_g)
    pltpu.async_copy(table_hbm.at[xf_all.at[pl.ds(off + _HALF, _P - _HALF)]],
                     rows_v[k].at[pl.ds(_HALF, _P - _HALF)], sem_g)

  def drain_gather(k):
    pltpu.make_async_copy(table_hbm.at[xf_all.at[pl.ds(0, _HALF)]],
                          rows_v[k].at[pl.ds(0, _HALF)], sem_g).wait()
    pltpu.make_async_copy(table_hbm.at[xf_all.at[pl.ds(0, _HALF)]],
                          rows_v[k].at[pl.ds(_HALF, _P - _HALF)],
                          sem_g).wait()

  def issue_flush(i, half):
    row0 = (wbase + i) * _ROWS_PER_B + half * _TROWS
    pltpu.async_copy(t_v[half], out_hbm.at[pl.ds(row0, _TROWS)], sem_f)

  def drain_flush(kt):
    pltpu.make_async_copy(t_v[kt], out_hbm.at[pl.ds(0, _TROWS)], sem_f).wait()

  def compute_f(i):
    off = i * _P
    for t in range(_PCHUNKS):
      p0 = t * _LANES
      xv = xf_all[pl.ds(off + p0, _LANES)]
      mv = m_all[pl.ds(off + p0, _LANES)]
      f_v[pl.ds(p0, _LANES)] = jnp.where(xv != 0, mv, zeros)

  def compute(i, k, maybe_first):
    compute_f(i)
    for half in range(2):
      if not maybe_first:
        drain_flush(half)
      else:
        @pl.when(i >= 1)
        def _():
          drain_flush(half)

      for cbl in range(_CB // 2):
        cb = half * (_CB // 2) + cbl
        c0 = cb * _LANES
        cboff = cbl * 32

        @plsc.parallel_loop(0, _PCHUNKS)
        def _t_loop(t):
          p0 = t * _LANES
          praw = p0 + lanes
          pmask = praw < _P
          idx_p = jnp.minimum(praw, _P - 1)
          prow = (idx_p >> 7) << 3
          pcol = idx_p & 127
          fch = f_v[pl.ds(p0, _LANES)]
          for d in range(_LANES):
            perm = (lanes + d) & (_LANES - 1)
            crow = ((perm >> 3) << 4) + (perm & 7) + cboff
            v = plsc.load_gather(rows_v[k], [idx_p, c0 + perm])
            plsc.store_scatter(t_v[half], [prow + crow, pcol], v * fch,
                               mask=pmask)

      issue_flush(i, half)

  issue_gather(0, 0)

  @pl.loop(0, _BPW // 2)
  def _b_loop(j):
    i0 = 2 * j
    issue_gather(i0 + 1, 1)
    drain_gather(0)
    compute(i0, 0, True)

    @pl.when(j < _BPW // 2 - 1)
    def _():
      issue_gather(i0 + 2, 0)
    drain_gather(1)
    compute(i0 + 1, 1, False)

  drain_flush(0)
  drain_flush(1)


@jax.jit
def _emb(xp, mp, table):
  mesh = plsc.VectorSubcoreMesh(core_axis_name="c", subcore_axis_name="s",
                                num_cores=_NC, num_subcores=_NS)
  return pl.kernel(
      _body,
      out_type=jax.ShapeDtypeStruct((_B * _ROWS_PER_B, 128), jnp.float32),
      mesh=mesh,
      compiler_params=pltpu.CompilerParams(needs_layout_passes=False,
                                           disable_bounds_checks=True),
      scratch_types=[
          pltpu.VMEM((_BPW * _P + _LANES,), jnp.int32),
          pltpu.VMEM((_BPW * _P + _LANES,), jnp.float32),
          pltpu.VMEM((_PCHUNKS * _LANES,), jnp.float32),
          pltpu.VMEM((_P, _C), jnp.float32),
          pltpu.VMEM((_P, _C), jnp.float32),
          pltpu.VMEM((_TROWS, 128), jnp.float32),
          pltpu.VMEM((_TROWS, 128), jnp.float32),
          pltpu.SemaphoreType.DMA,
          pltpu.SemaphoreType.DMA,
      ],
  )(xp, mp, table)


def kernel(x, mask, table):
  o = _emb(x.reshape(-1), mask.reshape(-1), table)
  o5 = o.reshape(_B, _C // 8, _PT, 8, 128)
  return o5.transpose(0, 1, 3, 2, 4).reshape(_B, _C, _PT * 128)[:, :, :_P]


# speedup vs baseline: 1.5353x; 1.5353x over previous
import functools

import jax
import jax.numpy as jnp
from jax import lax
from jax.experimental import pallas as pl
from jax.experimental.pallas import tpu as pltpu
from jax.experimental.pallas import tpu_sc as plsc

_B, _P, _C = 1024, 200, 128
_NC, _NS = 2, 16
_NW = _NC * _NS
_BPW = _B // _NW
_LANES = 16
_PCHUNKS = (_P + _LANES - 1) // _LANES
_HALF = 104
_PT = 2
_ROWS_PER_B = (_C // 8) * _PT * 8
_CB = _C // _LANES
_TROWS = 32


def _body(xp_hbm, mp_hbm, table_hbm, out_hbm,
          xf_all, m_all, f_v, rows_v0, rows_v1, rows_v2, t_v,
          sem_g, sem_f):
  rows_v = (rows_v0, rows_v1, rows_v2)
  cid = lax.axis_index("c")
  sid = lax.axis_index("s")
  wid = sid * _NC + cid
  wbase = wid * _BPW

  lanes = lax.iota(jnp.int32, _LANES)
  zeros = jnp.zeros((_LANES,), jnp.float32)

  pltpu.sync_copy(xp_hbm.at[pl.ds(wbase * _P, _BPW * _P)],
                  xf_all.at[pl.ds(0, _BPW * _P)])
  pltpu.sync_copy(mp_hbm.at[pl.ds(wbase * _P, _BPW * _P)],
                  m_all.at[pl.ds(0, _BPW * _P)])

  def issue_gather(i, k):
    off = i * _P
    pltpu.async_copy(table_hbm.at[xf_all.at[pl.ds(off, _HALF)]],
                     rows_v[k].at[pl.ds(0, _HALF)], sem_g)
    pltpu.async_copy(table_hbm.at[xf_all.at[pl.ds(off + _HALF, _P - _HALF)]],
                     rows_v[k].at[pl.ds(_HALF, _P - _HALF)], sem_g)

  def drain_gather(k):
    pltpu.make_async_copy(table_hbm.at[xf_all.at[pl.ds(0, _HALF)]],
                          rows_v[k].at[pl.ds(0, _HALF)], sem_g).wait()
    pltpu.make_async_copy(table_hbm.at[xf_all.at[pl.ds(0, _HALF)]],
                          rows_v[k].at[pl.ds(_HALF, _P - _HALF)],
                          sem_g).wait()

  def issue_flush(i, cb):
    row0 = (wbase + i) * _ROWS_PER_B + cb * _TROWS
    pltpu.async_copy(t_v.at[pl.ds(cb * _TROWS, _TROWS)],
                     out_hbm.at[pl.ds(row0, _TROWS)], sem_f)

  def drain_flush():
    pltpu.make_async_copy(t_v.at[pl.ds(0, _TROWS)],
                          out_hbm.at[pl.ds(0, _TROWS)], sem_f).wait()

  def compute_f(i):
    off = i * _P
    for t in range(_PCHUNKS):
      p0 = t * _LANES
      xv = xf_all[pl.ds(off + p0, _LANES)]
      mv = m_all[pl.ds(off + p0, _LANES)]
      f_v[pl.ds(p0, _LANES)] = jnp.where(xv != 0, mv, zeros)

  def compute(i, k, maybe_first):
    compute_f(i)

    @pl.loop(0, _CB)
    def _cb_loop(cb):
      c0 = cb * _LANES
      if maybe_first:
        @pl.when(i >= 1)
        def _():
          drain_flush()
      else:
        drain_flush()

      @plsc.parallel_loop(0, _PCHUNKS)
      def _t_loop(t):
        p0 = t * _LANES
        praw = p0 + lanes
        pmask = praw < _P
        idx_p = jnp.minimum(praw, _P - 1)
        prow = (idx_p >> 7) << 3
        pcol = idx_p & 127
        fch = f_v[pl.ds(p0, _LANES)]
        for d in range(_LANES):
          perm = (lanes + d) & (_LANES - 1)
          crow = ((perm >> 3) << 4) + (perm & 7)
          v = plsc.load_gather(rows_v[k], [idx_p, c0 + perm])
          plsc.store_scatter(t_v, [cb * _TROWS + prow + crow, pcol],
                             v * fch, mask=pmask)

      issue_flush(i, cb)

  issue_gather(0, 0)
  issue_gather(1, 1)

  @pl.loop(0, _BPW // 6)
  def _b_loop(j):
    i0 = 6 * j
    for r in range(6):
      i = i0 + r
      k = r % 3

      @pl.when(i + 2 < _BPW)
      def _():
        issue_gather(i + 2, (r + 2) % 3)
      drain_gather(k)
      compute(i, k, r == 0)

  for r in range(2):
    i = _BPW - 2 + r
    k = i % 3
    drain_gather(k)
    compute(i, k, False)

  @pl.loop(0, _CB)
  def _tail_drain(cb):
    drain_flush()


@jax.jit
def _emb(xp, mp, table):
  mesh = plsc.VectorSubcoreMesh(core_axis_name="c", subcore_axis_name="s",
                                num_cores=_NC, num_subcores=_NS)
  return pl.kernel(
      _body,
      out_type=jax.ShapeDtypeStruct((_B * _ROWS_PER_B, 128), jnp.float32),
      mesh=mesh,
      compiler_params=pltpu.CompilerParams(needs_layout_passes=False,
                                           disable_bounds_checks=True),
      scratch_types=[
          pltpu.VMEM((_BPW * _P + _LANES,), jnp.int32),
          pltpu.VMEM((_BPW * _P + _LANES,), jnp.float32),
          pltpu.VMEM((_PCHUNKS * _LANES,), jnp.float32),
          pltpu.VMEM((_P, _C), jnp.float32),
          pltpu.VMEM((_P, _C), jnp.float32),
          pltpu.VMEM((_P, _C), jnp.float32),
          pltpu.VMEM((_ROWS_PER_B, 128), jnp.float32),
          pltpu.SemaphoreType.DMA,
          pltpu.SemaphoreType.DMA,
      ],
  )(xp, mp, table)


def kernel(x, mask, table):
  o = _emb(x.reshape(-1), mask.reshape(-1), table)
  o5 = o.reshape(_B, _C // 8, _PT, 8, 128)
  return o5.transpose(0, 1, 3, 2, 4).reshape(_B, _C, _PT * 128)[:, :, :_P]
